# baseline (speedup 1.0000x reference)
import functools

import jax
import jax.numpy as jnp
from jax import lax
from jax.experimental import pallas as pl
from jax.experimental.pallas import tpu as pltpu
from jax.experimental.pallas import tpu_sc as plsc

_B, _L, _D = 4096, 50, 128
_N = _B * _L
_NC, _NS = 2, 16
_NW = _NC * _NS
_U = 4096
_UNITS = (10, 10, 10, 10, 10)
_C = 64
_BN = 8192


_NSLOT = 5


def _sc_gather_body(uidx_hbm, iidx_hbm, utab_hbm, itab_hbm, out_hbm,
                    *refs, pw):
    ubuf = refs[2:2 + _NSLOT]
    ibuf = refs[2 + _NSLOT:2 + 2 * _NSLOT]
    gsem = refs[2 + 2 * _NSLOT:2 + 3 * _NSLOT]
    wsem = refs[2 + 3 * _NSLOT:2 + 4 * _NSLOT]
    uidx_v, iidx_v = refs[0], refs[1]

    wid = lax.axis_index("s") * _NC + lax.axis_index("c")
    base = pl.multiple_of(wid * pw, pw)
    pltpu.sync_copy(uidx_hbm.at[pl.ds(base, pw)], uidx_v)
    pltpu.sync_copy(iidx_hbm.at[pl.ds(base, pw)], iidx_v)

    ng = pw // _C

    def fire(g, s):
        off = g * _C
        cu = pltpu.async_copy(utab_hbm.at[uidx_v.at[pl.ds(off, _C)]],
                              ubuf[s], gsem[s])
        ci = pltpu.async_copy(itab_hbm.at[iidx_v.at[pl.ds(off, _C)]],
                              ibuf[s], gsem[s])
        return cu, ci

    inflight = [None] * _NSLOT
    wb = [None] * _NSLOT
    for p in range(min(_NSLOT - 1, ng)):
        inflight[p] = fire(p, p)
    for g in range(ng):
        s = g % _NSLOT
        cu, ci = inflight[s]
        cu.wait()
        ci.wait()
        nxt = g + _NSLOT - 1
        if nxt < ng:
            s2 = nxt % _NSLOT
            if wb[s2] is not None:
                for w in wb[s2]:
                    w.wait()
                wb[s2] = None
            inflight[s2] = fire(nxt, s2)
        dst = out_hbm.at[pl.ds(base + g * _C, _C)]
        wu = pltpu.async_copy(ubuf[s], dst.at[:, pl.ds(0, _D)], wsem[s])
        wi = pltpu.async_copy(ibuf[s], dst.at[:, pl.ds(_D, _D)], wsem[s])
        wb[s] = (wu, wi)
    for s in range(_NSLOT):
        if wb[s] is not None:
            for w in wb[s]:
                w.wait()


def _make_sc_gather(nk):
    pw = nk // _NW
    mesh = plsc.VectorSubcoreMesh(core_axis_name="c", subcore_axis_name="s")
    return functools.partial(
        pl.kernel,
        mesh=mesh,
        out_type=jax.ShapeDtypeStruct((nk, 2 * _D), jnp.float32),
        scratch_types=(
            [pltpu.VMEM((pw,), jnp.int32)] * 2
            + [pltpu.VMEM((_C, _D), jnp.float32)] * (2 * _NSLOT)
            + [pltpu.SemaphoreType.DMA] * (2 * _NSLOT)
        ),
    )(functools.partial(_sc_gather_body, pw=pw))


_DN = (((1,), (1,)), ((), ()))


def _mlp_body(emb_ref, w1_ref, b1_ref, w2_ref, b2_ref,
              w3_ref, b3_ref, w4_ref, b4_ref, out_ref):
    h = lax.dot_general(emb_ref[...], w1_ref[...], _DN,
                        preferred_element_type=jnp.float32)
    h = jax.nn.relu(h + b1_ref[...])
    h = jax.nn.relu(lax.dot_general(h, w2_ref[...], _DN,
                                    preferred_element_type=jnp.float32)
                    + b2_ref[...])
    h = jax.nn.relu(lax.dot_general(h, w3_ref[...], _DN,
                                    preferred_element_type=jnp.float32)
                    + b3_ref[...])
    logit = jnp.sum(h * w4_ref[...], axis=1) + b4_ref[0, 0]
    out_ref[...] = jax.nn.sigmoid(logit).reshape(out_ref.shape)


def _mlp(emb, w1, b1, w2, b2, w3, b3, w4, b4):
    nk = emb.shape[0]
    grid = (nk // _BN,)
    full = lambda r, c: pl.BlockSpec((r, c), lambda n: (0, 0))
    return pl.pallas_call(
        _mlp_body,
        grid=grid,
        in_specs=[
            pl.BlockSpec((_BN, 2 * _D), lambda n: (n, 0)),
            full(256, 256),
            full(1, 256),
            full(128, 256),
            full(1, 128),
            full(64, 128),
            full(1, 64),
            full(1, 64),
            full(1, 1),
        ],
        out_specs=pl.BlockSpec((_BN // 128, 128), lambda n: (n, 0)),
        out_shape=jax.ShapeDtypeStruct((nk // 128, 128), jnp.float32),
    )(emb, w1, b1, w2, b2, w3, b3, w4, b4)


def kernel(user_matrix, item_matrix, user_table, item_table,
           W1, b1, W2, b2, W3, b3, W4, b4):
    uidx = user_matrix.reshape(-1).astype(jnp.int32)
    iidx = item_matrix.reshape(-1).astype(jnp.int32)
    b1r, b2r, b3r = b1.reshape(1, -1), b2.reshape(1, -1), b3.reshape(1, -1)
    w4r, b4r = W4.reshape(1, -1), b4.reshape(1, 1)
    embs = []
    off = 0
    for units in _UNITS:
        nk = units * _U
        embs.append(_make_sc_gather(nk)(uidx[off:off + nk], iidx[off:off + nk],
                                        user_table, item_table))
        off += nk
    outs = [_mlp(emb, W1, b1r, W2, b2r, W3, b3r, w4r, b4r) for emb in embs]
    return jnp.concatenate(outs, axis=0).reshape(_B, _L)


# speedup vs baseline: 1.0191x; 1.0191x over previous
import functools

import jax
import jax.numpy as jnp
from jax import lax
from jax.experimental import pallas as pl
from jax.experimental.pallas import tpu as pltpu
from jax.experimental.pallas import tpu_sc as plsc

_B, _L, _D = 4096, 50, 128
_N = _B * _L
_NC, _NS = 2, 16
_NW = _NC * _NS
_U = 4096
_UNITS = (10, 10, 10, 10, 10)
_C = 128
_BN = 8192


_NSLOT = 3


def _sc_gather_body(uidx_hbm, iidx_hbm, utab_hbm, itab_hbm, out_hbm,
                    *refs, pw):
    ubuf = refs[2:2 + _NSLOT]
    ibuf = refs[2 + _NSLOT:2 + 2 * _NSLOT]
    gsem = refs[2 + 2 * _NSLOT:2 + 3 * _NSLOT]
    wsem = refs[2 + 3 * _NSLOT:2 + 4 * _NSLOT]
    uidx_v, iidx_v = refs[0], refs[1]

    wid = lax.axis_index("s") * _NC + lax.axis_index("c")
    base = pl.multiple_of(wid * pw, pw)
    pltpu.sync_copy(uidx_hbm.at[pl.ds(base, pw)], uidx_v)
    pltpu.sync_copy(iidx_hbm.at[pl.ds(base, pw)], iidx_v)

    ng = pw // _C

    def fire(g, s):
        off = g * _C
        cu = pltpu.async_copy(utab_hbm.at[uidx_v.at[pl.ds(off, _C)]],
                              ubuf[s], gsem[s])
        ci = pltpu.async_copy(itab_hbm.at[iidx_v.at[pl.ds(off, _C)]],
                              ibuf[s], gsem[s])
        return cu, ci

    inflight = [None] * _NSLOT
    wb = [None] * _NSLOT
    for p in range(min(_NSLOT - 1, ng)):
        inflight[p] = fire(p, p)
    for g in range(ng):
        s = g % _NSLOT
        cu, ci = inflight[s]
        cu.wait()
        ci.wait()
        nxt = g + _NSLOT - 1
        if nxt < ng:
            s2 = nxt % _NSLOT
            if wb[s2] is not None:
                for w in wb[s2]:
                    w.wait()
                wb[s2] = None
            inflight[s2] = fire(nxt, s2)
        dst = out_hbm.at[pl.ds(base + g * _C, _C)]
        wu = pltpu.async_copy(ubuf[s], dst.at[:, pl.ds(0, _D)], wsem[s])
        wi = pltpu.async_copy(ibuf[s], dst.at[:, pl.ds(_D, _D)], wsem[s])
        wb[s] = (wu, wi)
    for s in range(_NSLOT):
        if wb[s] is not None:
            for w in wb[s]:
                w.wait()


def _make_sc_gather(nk):
    pw = nk // _NW
    mesh = plsc.VectorSubcoreMesh(core_axis_name="c", subcore_axis_name="s")
    return functools.partial(
        pl.kernel,
        mesh=mesh,
        out_type=jax.ShapeDtypeStruct((nk, 2 * _D), jnp.float32),
        scratch_types=(
            [pltpu.VMEM((pw,), jnp.int32)] * 2
            + [pltpu.VMEM((_C, _D), jnp.float32)] * (2 * _NSLOT)
            + [pltpu.SemaphoreType.DMA] * (2 * _NSLOT)
        ),
    )(functools.partial(_sc_gather_body, pw=pw))


_DN = (((1,), (1,)), ((), ()))


def _mlp_body(emb_ref, w1_ref, b1_ref, w2_ref, b2_ref,
              w3_ref, b3_ref, w4_ref, b4_ref, out_ref):
    h = lax.dot_general(emb_ref[...], w1_ref[...], _DN,
                        preferred_element_type=jnp.float32)
    h = jax.nn.relu(h + b1_ref[...])
    h = jax.nn.relu(lax.dot_general(h, w2_ref[...], _DN,
                                    preferred_element_type=jnp.float32)
                    + b2_ref[...])
    h = jax.nn.relu(lax.dot_general(h, w3_ref[...], _DN,
                                    preferred_element_type=jnp.float32)
                    + b3_ref[...])
    logit = jnp.sum(h * w4_ref[...], axis=1) + b4_ref[0, 0]
    out_ref[...] = jax.nn.sigmoid(logit).reshape(out_ref.shape)


def _mlp(emb, w1, b1, w2, b2, w3, b3, w4, b4):
    nk = emb.shape[0]
    grid = (nk // _BN,)
    full = lambda r, c: pl.BlockSpec((r, c), lambda n: (0, 0))
    return pl.pallas_call(
        _mlp_body,
        grid=grid,
        in_specs=[
            pl.BlockSpec((_BN, 2 * _D), lambda n: (n, 0)),
            full(256, 256),
            full(1, 256),
            full(128, 256),
            full(1, 128),
            full(64, 128),
            full(1, 64),
            full(1, 64),
            full(1, 1),
        ],
        out_specs=pl.BlockSpec((_BN // 128, 128), lambda n: (n, 0)),
        out_shape=jax.ShapeDtypeStruct((nk // 128, 128), jnp.float32),
    )(emb, w1, b1, w2, b2, w3, b3, w4, b4)


def kernel(user_matrix, item_matrix, user_table, item_table,
           W1, b1, W2, b2, W3, b3, W4, b4):
    uidx = user_matrix.reshape(-1).astype(jnp.int32)
    iidx = item_matrix.reshape(-1).astype(jnp.int32)
    b1r, b2r, b3r = b1.reshape(1, -1), b2.reshape(1, -1), b3.reshape(1, -1)
    w4r, b4r = W4.reshape(1, -1), b4.reshape(1, 1)
    embs = []
    off = 0
    for units in _UNITS:
        nk = units * _U
        embs.append(_make_sc_gather(nk)(uidx[off:off + nk], iidx[off:off + nk],
                                        user_table, item_table))
        off += nk
    outs = [_mlp(emb, W1, b1r, W2, b2r, W3, b3r, w4r, b4r) for emb in embs]
    return jnp.concatenate(outs, axis=0).reshape(_B, _L)
